# baseline (speedup 1.0000x reference)
import functools

import jax
import jax.numpy as jnp
from jax import lax
from jax.experimental import pallas as pl
from jax.experimental.pallas import tpu as pltpu
from jax.experimental.pallas import tpu_sc as plsc

_N = 10000
_E = 320000
_NC = 2
_NS = 16
_F = 160
_NPAD = 10112
_EC = _E // (_NC * _NS)
_CH = 80
_NCHUNK = _EC // _CH
_NST = 5
_SC_CH = _NCHUNK // _NST
_NB = _NPAD // _NS

_PREC = lax.Precision.DEFAULT

_mesh = plsc.VectorSubcoreMesh(core_axis_name="c", subcore_axis_name="s")


@functools.partial(
    pl.kernel,
    out_type=jax.ShapeDtypeStruct((_NC * _NPAD, 16), jnp.float32),
    mesh=_mesh,
    scratch_types=[
        pltpu.VMEM((_NCHUNK, _CH), jnp.int32),
        pltpu.VMEM((_CH, 16), jnp.float32),
        pltpu.VMEM_SHARED((_NPAD, 16), jnp.float32),
        pltpu.SemaphoreType.DMA,
    ],
    compiler_params=pltpu.CompilerParams(use_tc_tiling_on_sc=False),
)
def _deg_kernel(col_hbm, ones_hbm, zeros_hbm, deg_out, cstage_v, ones_v, hist,
                sem):
    cid = lax.axis_index("c")
    sid = lax.axis_index("s")

    pltpu.sync_copy(ones_hbm, ones_v)
    pltpu.sync_copy(zeros_hbm, hist.at[pl.ds(sid * _NB, _NB)])
    plsc.subcore_barrier()

    sbase = (cid * _NS + sid) * _NCHUNK
    pltpu.sync_copy(col_hbm.at[pl.ds(sbase, _NCHUNK)], cstage_v)

    @pl.loop(0, _NCHUNK)
    def _fire(c):
        pltpu.async_copy(ones_v, hist.at[cstage_v.at[c]], sem, add=True)

    @pl.loop(0, _NCHUNK)
    def _drain(c):
        pltpu.make_async_copy(ones_v, hist.at[cstage_v.at[0]], sem).wait()

    plsc.subcore_barrier()
    pltpu.sync_copy(hist.at[pl.ds(sid * _NB, _NB)],
                    deg_out.at[pl.ds(cid * _NPAD + sid * _NB, _NB)])


def _hs_body(x_ref, dpa_ref, dpb_ref, wc1_ref, wc2_ref, hs_ref):
    r = x_ref.shape[0]
    deg = 1.0 + dpa_ref[:, 0:1] + dpb_ref[:, 0:1]
    dis = lax.rsqrt(deg)
    h1 = jnp.dot(x_ref[:, :100], wc1_ref[...],
                 precision=_PREC,
                 preferred_element_type=jnp.float32)
    h2 = jnp.dot(x_ref[:, 100:150], wc2_ref[...],
                 precision=_PREC,
                 preferred_element_type=jnp.float32)
    pad = jnp.zeros((r, _F - 151), jnp.float32)
    hs_ref[...] = jnp.concatenate([h1 * dis, h2 * dis, dis, pad], axis=1)


_R = _NB
_G = _NPAD // _R

_hs_call = pl.pallas_call(
    _hs_body,
    grid=(_G,),
    in_specs=[
        pl.BlockSpec((_R, 150), lambda i: (i, 0)),
        pl.BlockSpec((_R, 16), lambda i: (i, 0)),
        pl.BlockSpec((_R, 16), lambda i: (i + _G, 0)),
        pl.BlockSpec((100, 100), lambda i: (0, 0)),
        pl.BlockSpec((50, 50), lambda i: (0, 0)),
    ],
    out_specs=pl.BlockSpec((_R, _F), lambda i: (i, 0)),
    out_shape=jax.ShapeDtypeStruct((_N, _F), jnp.float32),
)


@functools.partial(
    pl.kernel,
    out_type=jax.ShapeDtypeStruct((_NC * _NPAD, _F), jnp.float32),
    mesh=_mesh,
    scratch_types=[
        pltpu.VMEM((_SC_CH, _CH), jnp.int32),
        pltpu.VMEM((_SC_CH, _CH), jnp.int32),
        pltpu.VMEM((_CH, _F), jnp.float32),
        pltpu.VMEM((_CH, _F), jnp.float32),
        pltpu.VMEM_SHARED((_NPAD, _F), jnp.float32),
        pltpu.SemaphoreType.DMA,
        pltpu.SemaphoreType.DMA,
        pltpu.SemaphoreType.DMA,
        pltpu.SemaphoreType.DMA,
    ],
    compiler_params=pltpu.CompilerParams(use_tc_tiling_on_sc=False),
)
def _agg_kernel(hs_hbm, row_hbm, col_hbm, zero_hbm, acc_out,
                ridx_v, cidx_v, g0, g1, acc_sp, sg0, sg1, ss0, ss1):
    cid = lax.axis_index("c")
    sid = lax.axis_index("s")

    pltpu.sync_copy(zero_hbm, acc_sp.at[pl.ds(sid * _NB, _NB)])
    plsc.subcore_barrier()

    sbase = (cid * _NS + sid) * _NCHUNK

    gbufs = (g0, g1)
    gsems = (sg0, sg1)
    ssems = (ss0, ss1)

    def start_g(c, k):
        pltpu.async_copy(hs_hbm.at[ridx_v.at[c]], gbufs[k], gsems[k])

    def wait_g(k):
        pltpu.make_async_copy(hs_hbm.at[pl.ds(0, _CH)], gbufs[k],
                              gsems[k]).wait()

    def start_s(c, k):
        pltpu.async_copy(gbufs[k], acc_sp.at[cidx_v.at[c]], ssems[k],
                         add=True)

    def wait_s(k):
        pltpu.make_async_copy(gbufs[k], acc_sp.at[cidx_v.at[0]],
                              ssems[k]).wait()

    @pl.loop(0, _NST)
    def _stage(s):
        pltpu.sync_copy(row_hbm.at[pl.ds(sbase + s * _SC_CH, _SC_CH)],
                        ridx_v)
        pltpu.sync_copy(col_hbm.at[pl.ds(sbase + s * _SC_CH, _SC_CH)],
                        cidx_v)
        start_g(0, 0)
        start_g(1, 1)

        @pl.loop(0, (_SC_CH - 3) // 2)
        def _pair(i):
            c = 2 * i
            wait_g(0)
            start_s(c, 0)
            wait_g(1)
            start_s(c + 1, 1)
            wait_s(0)
            start_g(c + 2, 0)
            wait_s(1)
            start_g(c + 3, 1)

        c = _SC_CH - 3
        wait_g(0)
        start_s(c, 0)
        wait_g(1)
        start_s(c + 1, 1)
        wait_s(0)
        start_g(c + 2, 0)
        wait_s(1)
        wait_g(0)
        start_s(c + 2, 0)
        wait_s(0)

    plsc.subcore_barrier()
    pltpu.sync_copy(acc_sp.at[pl.ds(sid * _NB, _NB)],
                    acc_out.at[pl.ds(cid * _NPAD + sid * _NB, _NB)])


def _mlp_body(acca_ref, accb_ref, hs_ref, eps1_ref, eps2_ref,
              bc1_ref, bc2_ref,
              e1w1_ref, e1b1_ref, e1w2_ref, e1b2_ref, e1w3_ref, e1b3_ref,
              e2w1_ref, e2b1_ref, e2w2_ref, e2b2_ref, e2w3_ref, e2b3_ref,
              fcw_ref, fcb_ref,
              d1w1_ref, d1b1_ref, d1w2_ref, d1b2_ref, d1w3_ref, d1b3_ref,
              d2w1_ref, d2b1_ref, d2w2_ref, d2b2_ref, d2w3_ref, d2b3_ref,
              m1_ref, mu1_ref, lv1_ref, m2_ref, mu2_ref, lv2_ref, z_ref):
    def dot(a, w):
        return jnp.dot(a, w, precision=_PREC,
                       preferred_element_type=jnp.float32)

    def lrelu(v):
        return jnp.where(v >= 0, v, 0.01 * v)

    def sigmoid(v):
        return 1.0 / (1.0 + jnp.exp(-v))

    agg = acca_ref[...] + accb_ref[...] + hs_ref[...]
    dis = hs_ref[:, 150:151]
    h1 = agg[:, :100] * dis + bc1_ref[...]
    h2 = agg[:, 100:150] * dis + bc2_ref[...]

    o1 = lrelu(dot(h1, e1w1_ref[...]) + e1b1_ref[...])
    o1 = lrelu(dot(o1, e1w2_ref[...]) + e1b2_ref[...])
    o1 = dot(o1, e1w3_ref[...]) + e1b3_ref[...]
    o2 = lrelu(dot(h2, e2w1_ref[...]) + e2b1_ref[...])
    o2 = lrelu(dot(o2, e2w2_ref[...]) + e2b2_ref[...])
    o2 = dot(o2, e2w3_ref[...]) + e2b3_ref[...]

    mu1, lv1 = o1[:, :10], o1[:, 10:]
    mu2, lv2 = o2[:, :10], o2[:, 10:]
    z1 = mu1 + eps1_ref[...] * jnp.exp(0.5 * lv1)
    z2 = mu2 + eps2_ref[...] * jnp.exp(0.5 * lv2)
    zc = jnp.concatenate([z1, z2], axis=1)
    zz = jnp.maximum(dot(zc, fcw_ref[...]) + fcb_ref[...], 0.0)

    m1 = lrelu(dot(zz, d1w1_ref[...]) + d1b1_ref[...])
    m1 = lrelu(dot(m1, d1w2_ref[...]) + d1b2_ref[...])
    m1 = sigmoid(dot(m1, d1w3_ref[...]) + d1b3_ref[...])
    m2 = lrelu(dot(zz, d2w1_ref[...]) + d2b1_ref[...])
    m2 = lrelu(dot(m2, d2w2_ref[...]) + d2b2_ref[...])
    m2 = sigmoid(dot(m2, d2w3_ref[...]) + d2b3_ref[...])

    m1_ref[...] = m1
    mu1_ref[...] = mu1
    lv1_ref[...] = lv1
    m2_ref[...] = m2
    mu2_ref[...] = mu2
    lv2_ref[...] = lv2
    z_ref[...] = zz


def _full(shape):
    nd = len(shape)
    return pl.BlockSpec(shape, lambda i, _nd=nd: (0,) * _nd)


def _rows(f):
    return pl.BlockSpec((_R, f), lambda i: (i, 0))


_mlp_call = pl.pallas_call(
    _mlp_body,
    grid=(_G,),
    in_specs=[
        pl.BlockSpec((_R, _F), lambda i: (i, 0)),
        pl.BlockSpec((_R, _F), lambda i: (i + _G, 0)),
        _rows(_F),
        _rows(10), _rows(10),
        _full((1, 100)), _full((1, 50)),
        _full((100, 70)), _full((1, 70)),
        _full((70, 40)), _full((1, 40)),
        _full((40, 20)), _full((1, 20)),
        _full((50, 40)), _full((1, 40)),
        _full((40, 30)), _full((1, 30)),
        _full((30, 20)), _full((1, 20)),
        _full((20, 20)), _full((1, 20)),
        _full((20, 40)), _full((1, 40)),
        _full((40, 70)), _full((1, 70)),
        _full((70, 100)), _full((1, 100)),
        _full((20, 30)), _full((1, 30)),
        _full((30, 40)), _full((1, 40)),
        _full((40, 50)), _full((1, 50)),
    ],
    out_specs=[
        _rows(100), _rows(10), _rows(10),
        _rows(50), _rows(10), _rows(10), _rows(20),
    ],
    out_shape=[
        jax.ShapeDtypeStruct((_N, 100), jnp.float32),
        jax.ShapeDtypeStruct((_N, 10), jnp.float32),
        jax.ShapeDtypeStruct((_N, 10), jnp.float32),
        jax.ShapeDtypeStruct((_N, 50), jnp.float32),
        jax.ShapeDtypeStruct((_N, 10), jnp.float32),
        jax.ShapeDtypeStruct((_N, 10), jnp.float32),
        jax.ShapeDtypeStruct((_N, 20), jnp.float32),
    ],
)


def kernel(x, edge_index, Wc1, bc1, Wc2, bc2, e1w1, e1b1, e1w2, e1b2, e1w3,
           e1b3, e2w1, e2b1, e2w2, e2b2, e2w3, e2b3, fcw, fcb, d1w1, d1b1,
           d1w2, d1b2, d1w3, d1b3, d2w1, d2b1, d2w2, d2b2, d2w3, d2b3,
           eps1, eps2):
    row = edge_index[0].reshape(_E // _CH, _CH)
    col = edge_index[1].reshape(_E // _CH, _CH)

    ones = jnp.ones((_CH, 16), jnp.float32)
    zrows = jnp.zeros((_NB, 16), jnp.float32)
    degp = _deg_kernel(col, ones, zrows)

    hs = _hs_call(x, degp, degp, Wc1, Wc2)

    zeros = jnp.zeros((_NB, _F), jnp.float32)
    accs = _agg_kernel(hs, row, col, zeros)

    m1, mu1, lv1, m2, mu2, lv2, z = _mlp_call(
        accs, accs, hs, eps1, eps2,
        bc1.reshape(1, -1), bc2.reshape(1, -1),
        e1w1, e1b1.reshape(1, -1), e1w2, e1b2.reshape(1, -1),
        e1w3, e1b3.reshape(1, -1),
        e2w1, e2b1.reshape(1, -1), e2w2, e2b2.reshape(1, -1),
        e2w3, e2b3.reshape(1, -1),
        fcw, fcb.reshape(1, -1),
        d1w1, d1b1.reshape(1, -1), d1w2, d1b2.reshape(1, -1),
        d1w3, d1b3.reshape(1, -1),
        d2w1, d2b1.reshape(1, -1), d2w2, d2b2.reshape(1, -1),
        d2w3, d2b3.reshape(1, -1),
    )
    return (m1, mu1, lv1, m2, mu2, lv2, z)


# speedup vs baseline: 1.2346x; 1.2346x over previous
import functools

import jax
import jax.numpy as jnp
from jax import lax
from jax.experimental import pallas as pl
from jax.experimental.pallas import tpu as pltpu
from jax.experimental.pallas import tpu_sc as plsc

_N = 10000
_E = 320000
_NC = 2
_NS = 16
_F = 160
_NPAD = 10112
_EC = _E // (_NC * _NS)
_CH = 80
_NCHUNK = _EC // _CH
_NST = 5
_SC_CH = _NCHUNK // _NST
_NB = _NPAD // _NS

_PREC = lax.Precision.DEFAULT

_mesh = plsc.VectorSubcoreMesh(core_axis_name="c", subcore_axis_name="s")


@functools.partial(
    pl.kernel,
    out_type=jax.ShapeDtypeStruct((_NC * _NPAD, 16), jnp.float32),
    mesh=_mesh,
    scratch_types=[
        pltpu.VMEM((_NCHUNK, _CH), jnp.int32),
        pltpu.VMEM((_CH, 16), jnp.float32),
        pltpu.VMEM_SHARED((_NPAD, 16), jnp.float32),
        pltpu.SemaphoreType.DMA,
    ],
    compiler_params=pltpu.CompilerParams(use_tc_tiling_on_sc=False),
)
def _deg_kernel(ei_hbm, ones_hbm, zeros_hbm, deg_out, cstage_v, ones_v, hist,
                sem):
    cid = lax.axis_index("c")
    sid = lax.axis_index("s")

    pltpu.sync_copy(ones_hbm, ones_v)
    pltpu.sync_copy(zeros_hbm, hist.at[pl.ds(sid * _NB, _NB)])
    plsc.subcore_barrier()

    sbase = (cid * _NS + sid) * _NCHUNK
    pltpu.sync_copy(ei_hbm.at[1, pl.ds(sbase, _NCHUNK)], cstage_v)

    @pl.loop(0, _NCHUNK)
    def _fire(c):
        pltpu.async_copy(ones_v, hist.at[cstage_v.at[c]], sem, add=True)

    @pl.loop(0, _NCHUNK)
    def _drain(c):
        pltpu.make_async_copy(ones_v, hist.at[cstage_v.at[0]], sem).wait()

    plsc.subcore_barrier()
    pltpu.sync_copy(hist.at[pl.ds(sid * _NB, _NB)],
                    deg_out.at[pl.ds(cid * _NPAD + sid * _NB, _NB)])


def _hs_body(x_ref, dpa_ref, dpb_ref, wc1_ref, wc2_ref, hs_ref):
    r = x_ref.shape[0]
    deg = 1.0 + dpa_ref[:, 0:1] + dpb_ref[:, 0:1]
    dis = lax.rsqrt(deg)
    h1 = jnp.dot(x_ref[:, :100], wc1_ref[...],
                 precision=_PREC,
                 preferred_element_type=jnp.float32)
    h2 = jnp.dot(x_ref[:, 100:150], wc2_ref[...],
                 precision=_PREC,
                 preferred_element_type=jnp.float32)
    pad = jnp.zeros((r, _F - 151), jnp.float32)
    hs_ref[...] = jnp.concatenate([h1 * dis, h2 * dis, dis, pad], axis=1)


_R = 2 * _NB
_G = _NPAD // _R

_hs_call = pl.pallas_call(
    _hs_body,
    grid=(_G,),
    in_specs=[
        pl.BlockSpec((_R, 150), lambda i: (i, 0)),
        pl.BlockSpec((_R, 16), lambda i: (i, 0)),
        pl.BlockSpec((_R, 16), lambda i: (i + _G, 0)),
        pl.BlockSpec((100, 100), lambda i: (0, 0)),
        pl.BlockSpec((50, 50), lambda i: (0, 0)),
    ],
    out_specs=pl.BlockSpec((_R, _F), lambda i: (i, 0)),
    out_shape=jax.ShapeDtypeStruct((_N, _F), jnp.float32),
)


@functools.partial(
    pl.kernel,
    out_type=jax.ShapeDtypeStruct((_NC * _NPAD, _F), jnp.float32),
    mesh=_mesh,
    scratch_types=[
        pltpu.VMEM((_SC_CH, _CH), jnp.int32),
        pltpu.VMEM((_SC_CH, _CH), jnp.int32),
        pltpu.VMEM((_CH, _F), jnp.float32),
        pltpu.VMEM((_CH, _F), jnp.float32),
        pltpu.VMEM_SHARED((_NPAD, _F), jnp.float32),
        pltpu.SemaphoreType.DMA,
        pltpu.SemaphoreType.DMA,
    ],
    compiler_params=pltpu.CompilerParams(use_tc_tiling_on_sc=False),
)
def _agg_kernel(hs_hbm, ei_hbm, zero_hbm, acc_out,
                ridx_v, cidx_v, g0, g1, acc_sp, sg0, sg1):
    cid = lax.axis_index("c")
    sid = lax.axis_index("s")

    pltpu.sync_copy(zero_hbm, acc_sp.at[pl.ds(sid * _NB, _NB)])
    plsc.subcore_barrier()

    sbase = (cid * _NS + sid) * _NCHUNK

    gbufs = (g0, g1)
    gsems = (sg0, sg1)

    def start_g(c, k):
        pltpu.async_copy(hs_hbm.at[ridx_v.at[c]], gbufs[k], gsems[k])

    def wait_g(k):
        pltpu.make_async_copy(hs_hbm.at[pl.ds(0, _CH)], gbufs[k],
                              gsems[k]).wait()

    def sync_s(c, k):
        pltpu.sync_copy(gbufs[k], acc_sp.at[cidx_v.at[c]], add=True)

    @pl.loop(0, _NST)
    def _stage(s):
        pltpu.sync_copy(ei_hbm.at[0, pl.ds(sbase + s * _SC_CH, _SC_CH)],
                        ridx_v)
        pltpu.sync_copy(ei_hbm.at[1, pl.ds(sbase + s * _SC_CH, _SC_CH)],
                        cidx_v)
        start_g(0, 0)

        @pl.loop(0, (_SC_CH - 1) // 2)
        def _pair(i):
            c1 = 2 * i + 1
            start_g(c1, 1)
            wait_g(0)
            sync_s(2 * i, 0)
            start_g(c1 + 1, 0)
            wait_g(1)
            sync_s(c1, 1)

        wait_g(0)
        sync_s(_SC_CH - 1, 0)

    plsc.subcore_barrier()
    pltpu.sync_copy(acc_sp.at[pl.ds(sid * _NB, _NB)],
                    acc_out.at[pl.ds(cid * _NPAD + sid * _NB, _NB)])


def _mlp_body(acca_ref, accb_ref, hs_ref, eps1_ref, eps2_ref,
              bc1_ref, bc2_ref,
              e1w1_ref, e1b1_ref, e1w2_ref, e1b2_ref, e1w3_ref, e1b3_ref,
              e2w1_ref, e2b1_ref, e2w2_ref, e2b2_ref, e2w3_ref, e2b3_ref,
              fcw_ref, fcb_ref,
              d1w1_ref, d1b1_ref, d1w2_ref, d1b2_ref, d1w3_ref, d1b3_ref,
              d2w1_ref, d2b1_ref, d2w2_ref, d2b2_ref, d2w3_ref, d2b3_ref,
              m1_ref, mu1_ref, lv1_ref, m2_ref, mu2_ref, lv2_ref, z_ref):
    def dot(a, w):
        return jnp.dot(a, w, precision=_PREC,
                       preferred_element_type=jnp.float32)

    def lrelu(v):
        return jnp.where(v >= 0, v, 0.01 * v)

    def sigmoid(v):
        return 1.0 / (1.0 + jnp.exp(-v))

    agg = acca_ref[...] + accb_ref[...] + hs_ref[...]
    dis = hs_ref[:, 150:151]
    h1 = agg[:, :100] * dis + bc1_ref[...]
    h2 = agg[:, 100:150] * dis + bc2_ref[...]

    o1 = lrelu(dot(h1, e1w1_ref[...]) + e1b1_ref[...])
    o1 = lrelu(dot(o1, e1w2_ref[...]) + e1b2_ref[...])
    o1 = dot(o1, e1w3_ref[...]) + e1b3_ref[...]
    o2 = lrelu(dot(h2, e2w1_ref[...]) + e2b1_ref[...])
    o2 = lrelu(dot(o2, e2w2_ref[...]) + e2b2_ref[...])
    o2 = dot(o2, e2w3_ref[...]) + e2b3_ref[...]

    mu1, lv1 = o1[:, :10], o1[:, 10:]
    mu2, lv2 = o2[:, :10], o2[:, 10:]
    z1 = mu1 + eps1_ref[...] * jnp.exp(0.5 * lv1)
    z2 = mu2 + eps2_ref[...] * jnp.exp(0.5 * lv2)
    zc = jnp.concatenate([z1, z2], axis=1)
    zz = jnp.maximum(dot(zc, fcw_ref[...]) + fcb_ref[...], 0.0)

    m1 = lrelu(dot(zz, d1w1_ref[...]) + d1b1_ref[...])
    m1 = lrelu(dot(m1, d1w2_ref[...]) + d1b2_ref[...])
    m1 = sigmoid(dot(m1, d1w3_ref[...]) + d1b3_ref[...])
    m2 = lrelu(dot(zz, d2w1_ref[...]) + d2b1_ref[...])
    m2 = lrelu(dot(m2, d2w2_ref[...]) + d2b2_ref[...])
    m2 = sigmoid(dot(m2, d2w3_ref[...]) + d2b3_ref[...])

    m1_ref[...] = m1
    mu1_ref[...] = mu1
    lv1_ref[...] = lv1
    m2_ref[...] = m2
    mu2_ref[...] = mu2
    lv2_ref[...] = lv2
    z_ref[...] = zz


def _full(shape):
    nd = len(shape)
    return pl.BlockSpec(shape, lambda i, _nd=nd: (0,) * _nd)


def _rows(f):
    return pl.BlockSpec((_R, f), lambda i: (i, 0))


_mlp_call = pl.pallas_call(
    _mlp_body,
    grid=(_G,),
    in_specs=[
        pl.BlockSpec((_R, _F), lambda i: (i, 0)),
        pl.BlockSpec((_R, _F), lambda i: (i + _G, 0)),
        _rows(_F),
        _rows(10), _rows(10),
        _full((1, 100)), _full((1, 50)),
        _full((100, 70)), _full((1, 70)),
        _full((70, 40)), _full((1, 40)),
        _full((40, 20)), _full((1, 20)),
        _full((50, 40)), _full((1, 40)),
        _full((40, 30)), _full((1, 30)),
        _full((30, 20)), _full((1, 20)),
        _full((20, 20)), _full((1, 20)),
        _full((20, 40)), _full((1, 40)),
        _full((40, 70)), _full((1, 70)),
        _full((70, 100)), _full((1, 100)),
        _full((20, 30)), _full((1, 30)),
        _full((30, 40)), _full((1, 40)),
        _full((40, 50)), _full((1, 50)),
    ],
    out_specs=[
        _rows(100), _rows(10), _rows(10),
        _rows(50), _rows(10), _rows(10), _rows(20),
    ],
    out_shape=[
        jax.ShapeDtypeStruct((_N, 100), jnp.float32),
        jax.ShapeDtypeStruct((_N, 10), jnp.float32),
        jax.ShapeDtypeStruct((_N, 10), jnp.float32),
        jax.ShapeDtypeStruct((_N, 50), jnp.float32),
        jax.ShapeDtypeStruct((_N, 10), jnp.float32),
        jax.ShapeDtypeStruct((_N, 10), jnp.float32),
        jax.ShapeDtypeStruct((_N, 20), jnp.float32),
    ],
)


def kernel(x, edge_index, Wc1, bc1, Wc2, bc2, e1w1, e1b1, e1w2, e1b2, e1w3,
           e1b3, e2w1, e2b1, e2w2, e2b2, e2w3, e2b3, fcw, fcb, d1w1, d1b1,
           d1w2, d1b2, d1w3, d1b3, d2w1, d2b1, d2w2, d2b2, d2w3, d2b3,
           eps1, eps2):
    ei3 = edge_index.reshape(2, _E // _CH, _CH)

    ones = jnp.ones((_CH, 16), jnp.float32)
    zrows = jnp.zeros((_NB, 16), jnp.float32)
    degp = _deg_kernel(ei3, ones, zrows)

    hs = _hs_call(x, degp, degp, Wc1, Wc2)

    zeros = jnp.zeros((_NB, _F), jnp.float32)
    accs = _agg_kernel(hs, ei3, zeros)

    m1, mu1, lv1, m2, mu2, lv2, z = _mlp_call(
        accs, accs, hs, eps1, eps2,
        bc1.reshape(1, -1), bc2.reshape(1, -1),
        e1w1, e1b1.reshape(1, -1), e1w2, e1b2.reshape(1, -1),
        e1w3, e1b3.reshape(1, -1),
        e2w1, e2b1.reshape(1, -1), e2w2, e2b2.reshape(1, -1),
        e2w3, e2b3.reshape(1, -1),
        fcw, fcb.reshape(1, -1),
        d1w1, d1b1.reshape(1, -1), d1w2, d1b2.reshape(1, -1),
        d1w3, d1b3.reshape(1, -1),
        d2w1, d2b1.reshape(1, -1), d2w2, d2b2.reshape(1, -1),
        d2w3, d2b3.reshape(1, -1),
    )
    return (m1, mu1, lv1, m2, mu2, lv2, z)


# speedup vs baseline: 1.2394x; 1.0039x over previous
import functools

import jax
import jax.numpy as jnp
from jax import lax
from jax.experimental import pallas as pl
from jax.experimental.pallas import tpu as pltpu
from jax.experimental.pallas import tpu_sc as plsc

_N = 10000
_E = 320000
_NC = 2
_NS = 16
_F = 160
_NPAD = 10112
_EC = _E // (_NC * _NS)
_CH = 80
_NCHUNK = _EC // _CH
_NST = 5
_SC_CH = _NCHUNK // _NST
_NB = _NPAD // _NS

_PREC = lax.Precision.DEFAULT

_mesh = plsc.VectorSubcoreMesh(core_axis_name="c", subcore_axis_name="s")


@functools.partial(
    pl.kernel,
    out_type=jax.ShapeDtypeStruct((_NC * _NPAD, 16), jnp.float32),
    mesh=_mesh,
    scratch_types=[
        pltpu.VMEM((_NCHUNK, _CH), jnp.int32),
        pltpu.VMEM((_CH, 16), jnp.float32),
        pltpu.VMEM_SHARED((_NPAD, 16), jnp.float32),
        pltpu.SemaphoreType.DMA,
    ],
    compiler_params=pltpu.CompilerParams(use_tc_tiling_on_sc=False),
)
def _deg_kernel(ei_hbm, ones_hbm, zeros_hbm, deg_out, cstage_v, ones_v, hist,
                sem):
    cid = lax.axis_index("c")
    sid = lax.axis_index("s")

    pltpu.sync_copy(ones_hbm, ones_v)
    pltpu.sync_copy(zeros_hbm, hist.at[pl.ds(sid * _NB, _NB)])
    plsc.subcore_barrier()

    sbase = (cid * _NS + sid) * _NCHUNK
    pltpu.sync_copy(ei_hbm.at[1, pl.ds(sbase, _NCHUNK)], cstage_v)

    @pl.loop(0, _NCHUNK)
    def _fire(c):
        pltpu.async_copy(ones_v, hist.at[cstage_v.at[c]], sem, add=True)

    @pl.loop(0, _NCHUNK)
    def _drain(c):
        pltpu.make_async_copy(ones_v, hist.at[cstage_v.at[0]], sem).wait()

    plsc.subcore_barrier()
    pltpu.sync_copy(hist.at[pl.ds(sid * _NB, _NB)],
                    deg_out.at[pl.ds(cid * _NPAD + sid * _NB, _NB)])


def _hs_body(x_ref, dpa_ref, dpb_ref, wc1_ref, wc2_ref, hs_ref):
    r = x_ref.shape[0]
    deg = 1.0 + dpa_ref[:, 0:1] + dpb_ref[:, 0:1]
    dis = lax.rsqrt(deg)
    h1 = jnp.dot(x_ref[:, :100], wc1_ref[...],
                 precision=_PREC,
                 preferred_element_type=jnp.float32)
    h2 = jnp.dot(x_ref[:, 100:150], wc2_ref[...],
                 precision=_PREC,
                 preferred_element_type=jnp.float32)
    pad = jnp.zeros((r, _F - 151), jnp.float32)
    hs_ref[...] = jnp.concatenate([h1 * dis, h2 * dis, dis, pad], axis=1)


_R = 4 * _NB
_G = _NPAD // _R

_hs_call = pl.pallas_call(
    _hs_body,
    grid=(_G,),
    in_specs=[
        pl.BlockSpec((_R, 150), lambda i: (i, 0)),
        pl.BlockSpec((_R, 16), lambda i: (i, 0)),
        pl.BlockSpec((_R, 16), lambda i: (i + _G, 0)),
        pl.BlockSpec((100, 100), lambda i: (0, 0)),
        pl.BlockSpec((50, 50), lambda i: (0, 0)),
    ],
    out_specs=pl.BlockSpec((_R, _F), lambda i: (i, 0)),
    out_shape=jax.ShapeDtypeStruct((_N, _F), jnp.float32),
)


@functools.partial(
    pl.kernel,
    out_type=jax.ShapeDtypeStruct((_NC * _NPAD, _F), jnp.float32),
    mesh=_mesh,
    scratch_types=[
        pltpu.VMEM((_SC_CH, _CH), jnp.int32),
        pltpu.VMEM((_SC_CH, _CH), jnp.int32),
        pltpu.VMEM((_CH, _F), jnp.float32),
        pltpu.VMEM((_CH, _F), jnp.float32),
        pltpu.VMEM_SHARED((_NPAD, _F), jnp.float32),
        pltpu.SemaphoreType.DMA,
        pltpu.SemaphoreType.DMA,
    ],
    compiler_params=pltpu.CompilerParams(use_tc_tiling_on_sc=False),
)
def _agg_kernel(hs_hbm, ei_hbm, zero_hbm, acc_out,
                ridx_v, cidx_v, g0, g1, acc_sp, sg0, sg1):
    cid = lax.axis_index("c")
    sid = lax.axis_index("s")

    pltpu.sync_copy(zero_hbm, acc_sp.at[pl.ds(sid * _NB, _NB)])
    plsc.subcore_barrier()

    sbase = (cid * _NS + sid) * _NCHUNK

    gbufs = (g0, g1)
    gsems = (sg0, sg1)

    def start_g(c, k):
        pltpu.async_copy(hs_hbm.at[ridx_v.at[c]], gbufs[k], gsems[k])

    def wait_g(k):
        pltpu.make_async_copy(hs_hbm.at[pl.ds(0, _CH)], gbufs[k],
                              gsems[k]).wait()

    def sync_s(c, k):
        pltpu.sync_copy(gbufs[k], acc_sp.at[cidx_v.at[c]], add=True)

    @pl.loop(0, _NST)
    def _stage(s):
        pltpu.sync_copy(ei_hbm.at[0, pl.ds(sbase + s * _SC_CH, _SC_CH)],
                        ridx_v)
        pltpu.sync_copy(ei_hbm.at[1, pl.ds(sbase + s * _SC_CH, _SC_CH)],
                        cidx_v)
        start_g(0, 0)

        @pl.loop(0, (_SC_CH - 1) // 2)
        def _pair(i):
            c1 = 2 * i + 1
            start_g(c1, 1)
            wait_g(0)
            sync_s(2 * i, 0)
            start_g(c1 + 1, 0)
            wait_g(1)
            sync_s(c1, 1)

        wait_g(0)
        sync_s(_SC_CH - 1, 0)

    plsc.subcore_barrier()
    pltpu.sync_copy(acc_sp.at[pl.ds(sid * _NB, _NB)],
                    acc_out.at[pl.ds(cid * _NPAD + sid * _NB, _NB)])


def _mlp_body(acca_ref, accb_ref, hs_ref, eps1_ref, eps2_ref,
              bc1_ref, bc2_ref,
              e1w1_ref, e1b1_ref, e1w2_ref, e1b2_ref, e1w3_ref, e1b3_ref,
              e2w1_ref, e2b1_ref, e2w2_ref, e2b2_ref, e2w3_ref, e2b3_ref,
              fcw_ref, fcb_ref,
              d1w1_ref, d1b1_ref, d1w2_ref, d1b2_ref, d1w3_ref, d1b3_ref,
              d2w1_ref, d2b1_ref, d2w2_ref, d2b2_ref, d2w3_ref, d2b3_ref,
              m1_ref, mu1_ref, lv1_ref, m2_ref, mu2_ref, lv2_ref, z_ref):
    def dot(a, w):
        return jnp.dot(a, w, precision=_PREC,
                       preferred_element_type=jnp.float32)

    def lrelu(v):
        return jnp.where(v >= 0, v, 0.01 * v)

    def sigmoid(v):
        return 1.0 / (1.0 + jnp.exp(-v))

    agg = acca_ref[...] + accb_ref[...] + hs_ref[...]
    dis = hs_ref[:, 150:151]
    h1 = agg[:, :100] * dis + bc1_ref[...]
    h2 = agg[:, 100:150] * dis + bc2_ref[...]

    o1 = lrelu(dot(h1, e1w1_ref[...]) + e1b1_ref[...])
    o1 = lrelu(dot(o1, e1w2_ref[...]) + e1b2_ref[...])
    o1 = dot(o1, e1w3_ref[...]) + e1b3_ref[...]
    o2 = lrelu(dot(h2, e2w1_ref[...]) + e2b1_ref[...])
    o2 = lrelu(dot(o2, e2w2_ref[...]) + e2b2_ref[...])
    o2 = dot(o2, e2w3_ref[...]) + e2b3_ref[...]

    mu1, lv1 = o1[:, :10], o1[:, 10:]
    mu2, lv2 = o2[:, :10], o2[:, 10:]
    z1 = mu1 + eps1_ref[...] * jnp.exp(0.5 * lv1)
    z2 = mu2 + eps2_ref[...] * jnp.exp(0.5 * lv2)
    zc = jnp.concatenate([z1, z2], axis=1)
    zz = jnp.maximum(dot(zc, fcw_ref[...]) + fcb_ref[...], 0.0)

    m1 = lrelu(dot(zz, d1w1_ref[...]) + d1b1_ref[...])
    m1 = lrelu(dot(m1, d1w2_ref[...]) + d1b2_ref[...])
    m1 = sigmoid(dot(m1, d1w3_ref[...]) + d1b3_ref[...])
    m2 = lrelu(dot(zz, d2w1_ref[...]) + d2b1_ref[...])
    m2 = lrelu(dot(m2, d2w2_ref[...]) + d2b2_ref[...])
    m2 = sigmoid(dot(m2, d2w3_ref[...]) + d2b3_ref[...])

    m1_ref[...] = m1
    mu1_ref[...] = mu1
    lv1_ref[...] = lv1
    m2_ref[...] = m2
    mu2_ref[...] = mu2
    lv2_ref[...] = lv2
    z_ref[...] = zz


def _full(shape):
    nd = len(shape)
    return pl.BlockSpec(shape, lambda i, _nd=nd: (0,) * _nd)


def _rows(f):
    return pl.BlockSpec((_R, f), lambda i: (i, 0))


_mlp_call = pl.pallas_call(
    _mlp_body,
    grid=(_G,),
    in_specs=[
        pl.BlockSpec((_R, _F), lambda i: (i, 0)),
        pl.BlockSpec((_R, _F), lambda i: (i + _G, 0)),
        _rows(_F),
        _rows(10), _rows(10),
        _full((1, 100)), _full((1, 50)),
        _full((100, 70)), _full((1, 70)),
        _full((70, 40)), _full((1, 40)),
        _full((40, 20)), _full((1, 20)),
        _full((50, 40)), _full((1, 40)),
        _full((40, 30)), _full((1, 30)),
        _full((30, 20)), _full((1, 20)),
        _full((20, 20)), _full((1, 20)),
        _full((20, 40)), _full((1, 40)),
        _full((40, 70)), _full((1, 70)),
        _full((70, 100)), _full((1, 100)),
        _full((20, 30)), _full((1, 30)),
        _full((30, 40)), _full((1, 40)),
        _full((40, 50)), _full((1, 50)),
    ],
    out_specs=[
        _rows(100), _rows(10), _rows(10),
        _rows(50), _rows(10), _rows(10), _rows(20),
    ],
    out_shape=[
        jax.ShapeDtypeStruct((_N, 100), jnp.float32),
        jax.ShapeDtypeStruct((_N, 10), jnp.float32),
        jax.ShapeDtypeStruct((_N, 10), jnp.float32),
        jax.ShapeDtypeStruct((_N, 50), jnp.float32),
        jax.ShapeDtypeStruct((_N, 10), jnp.float32),
        jax.ShapeDtypeStruct((_N, 10), jnp.float32),
        jax.ShapeDtypeStruct((_N, 20), jnp.float32),
    ],
)


def kernel(x, edge_index, Wc1, bc1, Wc2, bc2, e1w1, e1b1, e1w2, e1b2, e1w3,
           e1b3, e2w1, e2b1, e2w2, e2b2, e2w3, e2b3, fcw, fcb, d1w1, d1b1,
           d1w2, d1b2, d1w3, d1b3, d2w1, d2b1, d2w2, d2b2, d2w3, d2b3,
           eps1, eps2):
    ei3 = edge_index.reshape(2, _E // _CH, _CH)

    ones = jnp.ones((_CH, 16), jnp.float32)
    zrows = jnp.zeros((_NB, 16), jnp.float32)
    degp = _deg_kernel(ei3, ones, zrows)

    hs = _hs_call(x, degp, degp, Wc1, Wc2)

    zeros = jnp.zeros((_NB, _F), jnp.float32)
    accs = _agg_kernel(hs, ei3, zeros)

    m1, mu1, lv1, m2, mu2, lv2, z = _mlp_call(
        accs, accs, hs, eps1, eps2,
        bc1.reshape(1, -1), bc2.reshape(1, -1),
        e1w1, e1b1.reshape(1, -1), e1w2, e1b2.reshape(1, -1),
        e1w3, e1b3.reshape(1, -1),
        e2w1, e2b1.reshape(1, -1), e2w2, e2b2.reshape(1, -1),
        e2w3, e2b3.reshape(1, -1),
        fcw, fcb.reshape(1, -1),
        d1w1, d1b1.reshape(1, -1), d1w2, d1b2.reshape(1, -1),
        d1w3, d1b3.reshape(1, -1),
        d2w1, d2b1.reshape(1, -1), d2w2, d2b2.reshape(1, -1),
        d2w3, d2b3.reshape(1, -1),
    )
    return (m1, mu1, lv1, m2, mu2, lv2, z)


# speedup vs baseline: 1.2706x; 1.0252x over previous
import functools

import jax
import jax.numpy as jnp
from jax import lax
from jax.experimental import pallas as pl
from jax.experimental.pallas import tpu as pltpu
from jax.experimental.pallas import tpu_sc as plsc

_N = 10000
_E = 320000
_NC = 2
_NS = 16
_F = 160
_NPAD = 10112
_EC = _E // (_NC * _NS)
_CH = 80
_NCHUNK = _EC // _CH
_NST = 5
_SC_CH = _NCHUNK // _NST
_NB = _NPAD // _NS

_PREC = lax.Precision.DEFAULT

_mesh = plsc.VectorSubcoreMesh(core_axis_name="c", subcore_axis_name="s")


@functools.partial(
    pl.kernel,
    out_type=jax.ShapeDtypeStruct((_NC * _NPAD, 16), jnp.float32),
    mesh=_mesh,
    scratch_types=[
        pltpu.VMEM((_NCHUNK, _CH), jnp.int32),
        pltpu.VMEM((_CH, 16), jnp.float32),
        pltpu.VMEM_SHARED((_NPAD, 16), jnp.float32),
        pltpu.SemaphoreType.DMA,
    ],
    compiler_params=pltpu.CompilerParams(use_tc_tiling_on_sc=False),
)
def _deg_kernel(ei_hbm, ones_hbm, zeros_hbm, deg_out, cstage_v, ones_v, hist,
                sem):
    cid = lax.axis_index("c")
    sid = lax.axis_index("s")

    pltpu.sync_copy(ones_hbm, ones_v)
    pltpu.sync_copy(zeros_hbm, hist.at[pl.ds(sid * _NB, _NB)])
    plsc.subcore_barrier()

    sbase = (cid * _NS + sid) * _NCHUNK
    pltpu.sync_copy(ei_hbm.at[1, pl.ds(sbase, _NCHUNK)], cstage_v)

    @pl.loop(0, _NCHUNK)
    def _fire(c):
        pltpu.async_copy(ones_v, hist.at[cstage_v.at[c]], sem, add=True)

    @pl.loop(0, _NCHUNK)
    def _drain(c):
        pltpu.make_async_copy(ones_v, hist.at[cstage_v.at[0]], sem).wait()

    plsc.subcore_barrier()
    pltpu.sync_copy(hist.at[pl.ds(sid * _NB, _NB)],
                    deg_out.at[pl.ds(cid * _NPAD + sid * _NB, _NB)])


def _hs_body(x_ref, dpa_ref, dpb_ref, wc1_ref, wc2_ref, hs_ref):
    r = x_ref.shape[0]
    deg = 1.0 + dpa_ref[:, 0:1] + dpb_ref[:, 0:1]
    dis = lax.rsqrt(deg)
    h1 = jnp.dot(x_ref[:, :100], wc1_ref[...],
                 precision=_PREC,
                 preferred_element_type=jnp.float32)
    h2 = jnp.dot(x_ref[:, 100:150], wc2_ref[...],
                 precision=_PREC,
                 preferred_element_type=jnp.float32)
    pad = jnp.zeros((r, _F - 151), jnp.float32)
    hs_ref[...] = jnp.concatenate([h1 * dis, h2 * dis, dis, pad], axis=1)


_R = 4 * _NB
_G = _NPAD // _R

_hs_call = pl.pallas_call(
    _hs_body,
    grid=(_G,),
    in_specs=[
        pl.BlockSpec((_R, 150), lambda i: (i, 0)),
        pl.BlockSpec((_R, 16), lambda i: (i, 0)),
        pl.BlockSpec((_R, 16), lambda i: (i + _G, 0)),
        pl.BlockSpec((100, 100), lambda i: (0, 0)),
        pl.BlockSpec((50, 50), lambda i: (0, 0)),
    ],
    out_specs=pl.BlockSpec((_R, _F), lambda i: (i, 0)),
    out_shape=jax.ShapeDtypeStruct((_N, _F), jnp.float32),
)


@functools.partial(
    pl.kernel,
    out_type=jax.ShapeDtypeStruct((_NC * _NPAD, _F), jnp.float32),
    mesh=_mesh,
    scratch_types=[
        pltpu.VMEM((_SC_CH, _CH), jnp.int32),
        pltpu.VMEM((_SC_CH, _CH), jnp.int32),
        pltpu.VMEM((_CH, _F), jnp.float32),
        pltpu.VMEM((_CH, _F), jnp.float32),
        pltpu.VMEM_SHARED((_NPAD, _F), jnp.float32),
        pltpu.SemaphoreType.DMA,
        pltpu.SemaphoreType.DMA,
    ],
    compiler_params=pltpu.CompilerParams(use_tc_tiling_on_sc=False),
)
def _agg_kernel(hs_hbm, ei_hbm, acc_out,
                ridx_v, cidx_v, g0, g1, acc_sp, sg0, sg1):
    cid = lax.axis_index("c")
    sid = lax.axis_index("s")

    zeros16 = jnp.zeros((16,), jnp.float32)

    @pl.loop(0, _CH)
    def _z(r):
        for j in range(_F // 16):
            g0[r, pl.ds(j * 16, 16)] = zeros16

    for t in range(_NB // _CH):
        pltpu.sync_copy(g0, acc_sp.at[pl.ds(sid * _NB + t * _CH, _CH)])
    pltpu.sync_copy(g0.at[pl.ds(0, _NB - (_NB // _CH) * _CH)],
                    acc_sp.at[pl.ds(sid * _NB + (_NB // _CH) * _CH,
                                    _NB - (_NB // _CH) * _CH)])
    plsc.subcore_barrier()

    sbase = (cid * _NS + sid) * _NCHUNK

    gbufs = (g0, g1)
    gsems = (sg0, sg1)

    def start_g(c, k):
        pltpu.async_copy(hs_hbm.at[ridx_v.at[c]], gbufs[k], gsems[k])

    def wait_g(k):
        pltpu.make_async_copy(hs_hbm.at[pl.ds(0, _CH)], gbufs[k],
                              gsems[k]).wait()

    def sync_s(c, k):
        pltpu.sync_copy(gbufs[k], acc_sp.at[cidx_v.at[c]], add=True)

    @pl.loop(0, _NST)
    def _stage(s):
        pltpu.sync_copy(ei_hbm.at[0, pl.ds(sbase + s * _SC_CH, _SC_CH)],
                        ridx_v)
        pltpu.sync_copy(ei_hbm.at[1, pl.ds(sbase + s * _SC_CH, _SC_CH)],
                        cidx_v)
        start_g(0, 0)

        @pl.loop(0, (_SC_CH - 1) // 2)
        def _pair(i):
            c1 = 2 * i + 1
            start_g(c1, 1)
            wait_g(0)
            sync_s(2 * i, 0)
            start_g(c1 + 1, 0)
            wait_g(1)
            sync_s(c1, 1)

        wait_g(0)
        sync_s(_SC_CH - 1, 0)

    plsc.subcore_barrier()
    pltpu.sync_copy(acc_sp.at[pl.ds(sid * _NB, _NB)],
                    acc_out.at[pl.ds(cid * _NPAD + sid * _NB, _NB)])


def _mlp_body(acca_ref, accb_ref, hs_ref, eps1_ref, eps2_ref,
              bc1_ref, bc2_ref,
              e1w1_ref, e1b1_ref, e1w2_ref, e1b2_ref, e1w3_ref, e1b3_ref,
              e2w1_ref, e2b1_ref, e2w2_ref, e2b2_ref, e2w3_ref, e2b3_ref,
              fcw_ref, fcb_ref,
              d1w1_ref, d1b1_ref, d1w2_ref, d1b2_ref, d1w3_ref, d1b3_ref,
              d2w1_ref, d2b1_ref, d2w2_ref, d2b2_ref, d2w3_ref, d2b3_ref,
              m1_ref, mu1_ref, lv1_ref, m2_ref, mu2_ref, lv2_ref, z_ref):
    def dot(a, w):
        return jnp.dot(a, w, precision=_PREC,
                       preferred_element_type=jnp.float32)

    def lrelu(v):
        return jnp.where(v >= 0, v, 0.01 * v)

    def sigmoid(v):
        return 1.0 / (1.0 + jnp.exp(-v))

    agg = acca_ref[...] + accb_ref[...] + hs_ref[...]
    dis = hs_ref[:, 150:151]
    h1 = agg[:, :100] * dis + bc1_ref[...]
    h2 = agg[:, 100:150] * dis + bc2_ref[...]

    o1 = lrelu(dot(h1, e1w1_ref[...]) + e1b1_ref[...])
    o1 = lrelu(dot(o1, e1w2_ref[...]) + e1b2_ref[...])
    o1 = dot(o1, e1w3_ref[...]) + e1b3_ref[...]
    o2 = lrelu(dot(h2, e2w1_ref[...]) + e2b1_ref[...])
    o2 = lrelu(dot(o2, e2w2_ref[...]) + e2b2_ref[...])
    o2 = dot(o2, e2w3_ref[...]) + e2b3_ref[...]

    mu1, lv1 = o1[:, :10], o1[:, 10:]
    mu2, lv2 = o2[:, :10], o2[:, 10:]
    z1 = mu1 + eps1_ref[...] * jnp.exp(0.5 * lv1)
    z2 = mu2 + eps2_ref[...] * jnp.exp(0.5 * lv2)
    zc = jnp.concatenate([z1, z2], axis=1)
    zz = jnp.maximum(dot(zc, fcw_ref[...]) + fcb_ref[...], 0.0)

    m1 = lrelu(dot(zz, d1w1_ref[...]) + d1b1_ref[...])
    m1 = lrelu(dot(m1, d1w2_ref[...]) + d1b2_ref[...])
    m1 = sigmoid(dot(m1, d1w3_ref[...]) + d1b3_ref[...])
    m2 = lrelu(dot(zz, d2w1_ref[...]) + d2b1_ref[...])
    m2 = lrelu(dot(m2, d2w2_ref[...]) + d2b2_ref[...])
    m2 = sigmoid(dot(m2, d2w3_ref[...]) + d2b3_ref[...])

    m1_ref[...] = m1
    mu1_ref[...] = mu1
    lv1_ref[...] = lv1
    m2_ref[...] = m2
    mu2_ref[...] = mu2
    lv2_ref[...] = lv2
    z_ref[...] = zz


def _full(shape):
    nd = len(shape)
    return pl.BlockSpec(shape, lambda i, _nd=nd: (0,) * _nd)


def _rows(f):
    return pl.BlockSpec((_R, f), lambda i: (i, 0))


_mlp_call = pl.pallas_call(
    _mlp_body,
    grid=(_G,),
    in_specs=[
        pl.BlockSpec((_R, _F), lambda i: (i, 0)),
        pl.BlockSpec((_R, _F), lambda i: (i + _G, 0)),
        _rows(_F),
        _rows(10), _rows(10),
        _full((1, 100)), _full((1, 50)),
        _full((100, 70)), _full((1, 70)),
        _full((70, 40)), _full((1, 40)),
        _full((40, 20)), _full((1, 20)),
        _full((50, 40)), _full((1, 40)),
        _full((40, 30)), _full((1, 30)),
        _full((30, 20)), _full((1, 20)),
        _full((20, 20)), _full((1, 20)),
        _full((20, 40)), _full((1, 40)),
        _full((40, 70)), _full((1, 70)),
        _full((70, 100)), _full((1, 100)),
        _full((20, 30)), _full((1, 30)),
        _full((30, 40)), _full((1, 40)),
        _full((40, 50)), _full((1, 50)),
    ],
    out_specs=[
        _rows(100), _rows(10), _rows(10),
        _rows(50), _rows(10), _rows(10), _rows(20),
    ],
    out_shape=[
        jax.ShapeDtypeStruct((_N, 100), jnp.float32),
        jax.ShapeDtypeStruct((_N, 10), jnp.float32),
        jax.ShapeDtypeStruct((_N, 10), jnp.float32),
        jax.ShapeDtypeStruct((_N, 50), jnp.float32),
        jax.ShapeDtypeStruct((_N, 10), jnp.float32),
        jax.ShapeDtypeStruct((_N, 10), jnp.float32),
        jax.ShapeDtypeStruct((_N, 20), jnp.float32),
    ],
)


def kernel(x, edge_index, Wc1, bc1, Wc2, bc2, e1w1, e1b1, e1w2, e1b2, e1w3,
           e1b3, e2w1, e2b1, e2w2, e2b2, e2w3, e2b3, fcw, fcb, d1w1, d1b1,
           d1w2, d1b2, d1w3, d1b3, d2w1, d2b1, d2w2, d2b2, d2w3, d2b3,
           eps1, eps2):
    ei3 = edge_index.reshape(2, _E // _CH, _CH)

    ones = jnp.ones((_CH, 16), jnp.float32)
    zrows = jnp.zeros((_NB, 16), jnp.float32)
    degp = _deg_kernel(ei3, ones, zrows)

    hs = _hs_call(x, degp, degp, Wc1, Wc2)

    accs = _agg_kernel(hs, ei3)

    m1, mu1, lv1, m2, mu2, lv2, z = _mlp_call(
        accs, accs, hs, eps1, eps2,
        bc1.reshape(1, -1), bc2.reshape(1, -1),
        e1w1, e1b1.reshape(1, -1), e1w2, e1b2.reshape(1, -1),
        e1w3, e1b3.reshape(1, -1),
        e2w1, e2b1.reshape(1, -1), e2w2, e2b2.reshape(1, -1),
        e2w3, e2b3.reshape(1, -1),
        fcw, fcb.reshape(1, -1),
        d1w1, d1b1.reshape(1, -1), d1w2, d1b2.reshape(1, -1),
        d1w3, d1b3.reshape(1, -1),
        d2w1, d2b1.reshape(1, -1), d2w2, d2b2.reshape(1, -1),
        d2w3, d2b3.reshape(1, -1),
    )
    return (m1, mu1, lv1, m2, mu2, lv2, z)
